# Initial kernel scaffold; baseline (speedup 1.0000x reference)
import jax
import jax.numpy as jnp
from jax.experimental import pallas as pl


def kernel(node_type, num_inverted_predecessors, edge_index, batch, seq_embedding, emb_table, W1, b1, W2, b2, Wd, bd, Wp1, bp1, Wp2, bp2, Wv1, bv1, Wv2, bv2):
    raise NotImplementedError("write your pallas kernel here")


# speedup vs baseline: 1.1666x; 1.1666x over previous
import jax
import jax.numpy as jnp
from jax.experimental import pallas as pl

N = 100000
G = 16


def _head_body(final_ref, Wd_ref, bd_ref, Wp1_ref, bp1_ref, Wp2_ref, bp2_ref,
               Wv1_ref, bv1_ref, Wv2_ref, bv2_ref,
               logits_ref, policy_ref, value_ref):
    final = final_ref[...]
    hfc = jax.nn.leaky_relu(final @ Wd_ref[...] + bd_ref[...])
    p1 = jax.nn.leaky_relu(hfc @ Wp1_ref[...] + bp1_ref[...])
    v1 = jax.nn.leaky_relu(hfc @ Wv1_ref[...] + bv1_ref[...])
    logits = p1 @ Wp2_ref[...] + bp2_ref[...]
    logits_ref[...] = logits
    policy_ref[...] = jax.nn.softmax(logits, axis=1)
    value_ref[...] = jnp.tanh(v1 @ Wv2_ref[...] + bv2_ref[...])


def kernel(node_type, num_inverted_predecessors, edge_index, batch, seq_embedding,
           emb_table, W1, b1, W2, b2, Wd, bd, Wp1, bp1, Wp2, bp2, Wv1, bv1, Wv2, bv2):
    src, dst = edge_index[0], edge_index[1]
    ninv = num_inverted_predecessors
    c = node_type * 3 + ninv
    table9 = jnp.concatenate([
        emb_table[jnp.arange(9) // 3],
        (jnp.arange(9) % 3).astype(jnp.float32)[:, None]], axis=1)
    A = table9 @ W1
    deg = jnp.zeros((N,), jnp.float32).at[dst].add(1.0) + 1.0
    dinv = jax.lax.rsqrt(deg)
    t = jnp.zeros((N, 9), jnp.float32).at[dst, c[src]].add(dinv[src])
    T = dinv[:, None] * t + (dinv * dinv)[:, None] * jax.nn.one_hot(c, 9, dtype=jnp.float32)
    h1 = jax.nn.relu(T @ A + b1)
    q = h1 * dinv[:, None]
    z = jnp.zeros((N, 32), jnp.float32).at[dst].add(q[src])
    h2 = (dinv[:, None] * z + (dinv * dinv)[:, None] * h1) @ W2 + b2
    counts = jax.ops.segment_sum(jnp.ones((N,), jnp.float32), batch, num_segments=G)
    mean_pool = jax.ops.segment_sum(h2, batch, num_segments=G) / jnp.maximum(counts, 1.0)[:, None]
    max_pool = jax.ops.segment_max(h2, batch, num_segments=G)
    aig = jnp.concatenate([mean_pool, max_pool], axis=1)
    aig = jnp.round(aig * 1000.0) / 1000.0
    final = jnp.concatenate([aig, seq_embedding], axis=1)

    logits, policy, value = pl.pallas_call(
        _head_body,
        out_shape=(
            jax.ShapeDtypeStruct((G, 7), jnp.float32),
            jax.ShapeDtypeStruct((G, 7), jnp.float32),
            jax.ShapeDtypeStruct((G, 1), jnp.float32),
        ),
    )(final, Wd, bd, Wp1, bp1, Wp2, bp2, Wv1, bv1, Wv2, bv2)
    return (logits, policy, value.reshape(-1), final, aig)


# speedup vs baseline: 63.8718x; 54.7496x over previous
import functools

import jax
import jax.numpy as jnp
from jax import lax
from jax.experimental import pallas as pl
from jax.experimental.pallas import tpu as pltpu
from jax.experimental.pallas import tpu_sc as plsc

N = 100000
NP = 100096
E = 6400000
G = 16
CHN = NP // 16
W = 2000
WZ = 800

_MESH = plsc.VectorSubcoreMesh(core_axis_name="c", subcore_axis_name="s")


def _deg_body(dst_hbm, ones_hbm, zeros_hbm, out0_hbm, out1_hbm,
              idx_v, ones_v, bz_v, deg_sh, sem):
    cid = lax.axis_index("c")
    sid = lax.axis_index("s")
    pltpu.sync_copy(zeros_hbm.at[pl.ds(sid * CHN, CHN)], bz_v)
    pltpu.sync_copy(bz_v, deg_sh.at[pl.ds(sid * CHN, CHN)])
    pltpu.sync_copy(ones_hbm, ones_v)
    plsc.subcore_barrier()

    chunk = E // 32
    base0 = cid * (E // 2) + sid * chunk

    def wbody(w, _):
        base = base0 + w * W
        pltpu.sync_copy(dst_hbm.at[pl.ds(base, W)], idx_v)
        pltpu.sync_copy(ones_v, deg_sh.at[idx_v], add=True)
        return 0

    lax.fori_loop(0, chunk // W, wbody, 0)
    plsc.subcore_barrier()

    pltpu.sync_copy(deg_sh.at[pl.ds(sid * CHN, CHN)], bz_v)

    @pl.when(cid == 0)
    def _():
        pltpu.sync_copy(bz_v, out0_hbm.at[pl.ds(sid * CHN, CHN)])

    @pl.when(cid == 1)
    def _():
        pltpu.sync_copy(bz_v, out1_hbm.at[pl.ds(sid * CHN, CHN)])


_deg_call = functools.partial(
    pl.kernel, _deg_body,
    out_type=(jax.ShapeDtypeStruct((NP,), jnp.float32),
              jax.ShapeDtypeStruct((NP,), jnp.float32)),
    mesh=_MESH,
    compiler_params=pltpu.CompilerParams(use_tc_tiling_on_sc=False),
    scratch_types=[
        pltpu.VMEM((W,), jnp.int32),
        pltpu.VMEM((W,), jnp.float32),
        pltpu.VMEM((CHN,), jnp.float32),
        pltpu.VMEM_SHARED((NP,), jnp.float32),
        pltpu.SemaphoreType.DMA,
    ],
)


def _tacc_body(src_hbm, dst_hbm, dinv_hbm, cns_hbm, zeros_hbm, out0_hbm, out1_hbm,
               sidx_v, didx_v, dval_v, cval_v, idx2_v, bzf_v, bdf_v, bdi_v,
               dinv_sh, cns_sh, t_sh, sem1, sem2):
    cid = lax.axis_index("c")
    sid = lax.axis_index("s")
    pltpu.sync_copy(dinv_hbm.at[pl.ds(sid * CHN, CHN)], bdf_v)
    pltpu.sync_copy(bdf_v, dinv_sh.at[pl.ds(sid * CHN, CHN)])
    pltpu.sync_copy(cns_hbm.at[pl.ds(sid * CHN, CHN)], bdi_v)
    pltpu.sync_copy(bdi_v, cns_sh.at[pl.ds(sid * CHN, CHN)])
    pltpu.sync_copy(zeros_hbm.at[pl.ds(0, CHN)], bzf_v)
    for k in range(9):
        pltpu.sync_copy(bzf_v, t_sh.at[pl.ds(sid * (CHN * 9) + k * CHN, CHN)])
    plsc.subcore_barrier()

    chunk = E // 32
    base0 = cid * (E // 2) + sid * chunk

    def wbody(w, _):
        base = base0 + w * W
        pltpu.sync_copy(src_hbm.at[pl.ds(base, W)], sidx_v)
        pltpu.sync_copy(dst_hbm.at[pl.ds(base, W)], didx_v)
        g1 = pltpu.async_copy(dinv_sh.at[sidx_v], dval_v, sem1)
        g2 = pltpu.async_copy(cns_sh.at[sidx_v], cval_v, sem2)
        g1.wait()
        g2.wait()

        def jbody(j, _):
            sl = pl.ds(j * 16, 16)
            idx2_v[sl] = didx_v[sl] * 9 + cval_v[sl]
            return 0

        lax.fori_loop(0, W // 16, jbody, 0)
        pltpu.sync_copy(dval_v, t_sh.at[idx2_v], add=True)
        return 0

    lax.fori_loop(0, chunk // W, wbody, 0)
    plsc.subcore_barrier()

    for k in range(9):
        off = sid * (CHN * 9) + k * CHN
        pltpu.sync_copy(t_sh.at[pl.ds(off, CHN)], bzf_v)

        @pl.when(cid == 0)
        def _():
            pltpu.sync_copy(bzf_v, out0_hbm.at[pl.ds(off, CHN)])

        @pl.when(cid == 1)
        def _():
            pltpu.sync_copy(bzf_v, out1_hbm.at[pl.ds(off, CHN)])


_tacc_call = functools.partial(
    pl.kernel, _tacc_body,
    out_type=(jax.ShapeDtypeStruct((NP * 9,), jnp.float32),
              jax.ShapeDtypeStruct((NP * 9,), jnp.float32)),
    mesh=_MESH,
    compiler_params=pltpu.CompilerParams(use_tc_tiling_on_sc=False),
    scratch_types=[
        pltpu.VMEM((W,), jnp.int32),
        pltpu.VMEM((W,), jnp.int32),
        pltpu.VMEM((W,), jnp.float32),
        pltpu.VMEM((W,), jnp.int32),
        pltpu.VMEM((W,), jnp.int32),
        pltpu.VMEM((CHN,), jnp.float32),
        pltpu.VMEM((CHN,), jnp.float32),
        pltpu.VMEM((CHN,), jnp.int32),
        pltpu.VMEM_SHARED((NP,), jnp.float32),
        pltpu.VMEM_SHARED((NP,), jnp.int32),
        pltpu.VMEM_SHARED((NP * 9,), jnp.float32),
        pltpu.SemaphoreType.DMA,
        pltpu.SemaphoreType.DMA,
    ],
)


def _zacc_body(src_hbm, dst_hbm, qA_hbm, qB_hbm, zeros_hbm, outA_hbm, outB_hbm,
               sidx_v, didx_v, rows_v, bq_v, q_sh, z_sh, sem):
    cid = lax.axis_index("c")
    sid = lax.axis_index("s")
    ZB = CHN // 8
    for k in range(8):
        off = sid * CHN + k * ZB

        @pl.when(cid == 0)
        def _():
            pltpu.sync_copy(qA_hbm.at[pl.ds(off, ZB)], bq_v)

        @pl.when(cid == 1)
        def _():
            pltpu.sync_copy(qB_hbm.at[pl.ds(off, ZB)], bq_v)

        pltpu.sync_copy(bq_v, q_sh.at[pl.ds(off, ZB)])
        pltpu.sync_copy(zeros_hbm.at[pl.ds(0, ZB)], bq_v)
        pltpu.sync_copy(bq_v, z_sh.at[pl.ds(off, ZB)])
    plsc.subcore_barrier()

    chunk = E // 16
    base0 = sid * chunk

    def wbody(w, _):
        base = base0 + w * WZ
        pltpu.sync_copy(src_hbm.at[pl.ds(base, WZ)], sidx_v)
        pltpu.sync_copy(dst_hbm.at[pl.ds(base, WZ)], didx_v)
        pltpu.async_copy(q_sh.at[sidx_v], rows_v, sem).wait()
        pltpu.sync_copy(rows_v, z_sh.at[didx_v], add=True)
        return 0

    lax.fori_loop(0, chunk // WZ, wbody, 0)
    plsc.subcore_barrier()

    for k in range(8):
        off = sid * CHN + k * ZB
        pltpu.sync_copy(z_sh.at[pl.ds(off, ZB)], bq_v)

        @pl.when(cid == 0)
        def _():
            pltpu.sync_copy(bq_v, outA_hbm.at[pl.ds(off, ZB)])

        @pl.when(cid == 1)
        def _():
            pltpu.sync_copy(bq_v, outB_hbm.at[pl.ds(off, ZB)])


_zacc_call = functools.partial(
    pl.kernel, _zacc_body,
    out_type=(jax.ShapeDtypeStruct((NP, 8), jnp.float32),
              jax.ShapeDtypeStruct((NP, 8), jnp.float32)),
    mesh=_MESH,
    compiler_params=pltpu.CompilerParams(use_tc_tiling_on_sc=False),
    scratch_types=[
        pltpu.VMEM((WZ,), jnp.int32),
        pltpu.VMEM((WZ,), jnp.int32),
        pltpu.VMEM((WZ, 8), jnp.float32),
        pltpu.VMEM((CHN // 8, 8), jnp.float32),
        pltpu.VMEM_SHARED((NP, 8), jnp.float32),
        pltpu.VMEM_SHARED((NP, 8), jnp.float32),
        pltpu.SemaphoreType.DMA,
    ],
)


def _prep_body(deg0_ref, deg1_ref, nt_ref, ninv_ref, dinv_ref, cns_ref):
    deg = deg0_ref[...] + deg1_ref[...] + 1.0
    dinv_ref[...] = lax.rsqrt(deg)
    cns_ref[...] = nt_ref[...] * 3 + ninv_ref[...]


def _h1q_body(t0_ref, t1_ref, dinv_ref, cns_ref, A_ref, b1_ref,
              h1_ref, qa_ref, qb_ref, qc_ref, qd_ref):
    dinv = dinv_ref[...]
    t = t0_ref[...] + t1_ref[...]
    onehot = (lax.broadcasted_iota(jnp.int32, t.shape, 1) == cns_ref[...]
              ).astype(jnp.float32)
    T9 = dinv * t + (dinv * dinv) * onehot
    T = jnp.concatenate([T9, jnp.zeros((T9.shape[0], 7), jnp.float32)], axis=1)
    h1 = jax.nn.relu(jnp.dot(T, A_ref[...], preferred_element_type=jnp.float32)
                     + b1_ref[...])
    h1_ref[...] = h1
    q = h1 * dinv
    qa_ref[...] = q[:, 0:8]
    qb_ref[...] = q[:, 8:16]
    qc_ref[...] = q[:, 16:24]
    qd_ref[...] = q[:, 24:32]


def _pool_body(za_ref, zb_ref, zc_ref, zd_ref, h1_ref, dinv_ref, batch_ref,
               W2_ref, b2_ref, aig_ref, sum_acc, cnt_acc, max_acc):
    i = pl.program_id(0)

    @pl.when(i == 0)
    def _():
        sum_acc[...] = jnp.zeros_like(sum_acc)
        cnt_acc[...] = jnp.zeros_like(cnt_acc)
        max_acc[...] = jnp.full_like(max_acc, -jnp.inf)

    dinv = dinv_ref[...]
    h1 = h1_ref[...]
    z = jnp.concatenate([za_ref[...], zb_ref[...], zc_ref[...], zd_ref[...]],
                        axis=1)
    u = dinv * z + (dinv * dinv) * h1
    h2 = jnp.dot(u, W2_ref[...], preferred_element_type=jnp.float32) + b2_ref[...]
    b = batch_ref[...]
    onehot = (lax.broadcasted_iota(jnp.int32, (h2.shape[0], G), 1) == b
              ).astype(jnp.float32)
    sum_acc[...] += jax.lax.dot_general(
        onehot, h2, (((0,), (0,)), ((), ())), preferred_element_type=jnp.float32)
    cnt_acc[...] += jnp.sum(onehot, axis=0)[:, None]
    for g in range(G):
        m = jnp.where(b == g, h2, -jnp.inf)
        max_acc[g, :] = jnp.maximum(max_acc[g, :], jnp.max(m, axis=0))

    @pl.when(i == pl.num_programs(0) - 1)
    def _():
        mean = sum_acc[...] / jnp.maximum(cnt_acc[...], 1.0)
        aig = jnp.concatenate([mean, max_acc[...]], axis=1)
        aig_ref[...] = jnp.round(aig * 1000.0) / 1000.0


def _head_body(final_ref, Wd_ref, bd_ref, Wp1_ref, bp1_ref, Wp2_ref, bp2_ref,
               Wv1_ref, bv1_ref, Wv2_ref, bv2_ref,
               logits_ref, policy_ref, value_ref):
    final = final_ref[...]
    hfc = jax.nn.leaky_relu(final @ Wd_ref[...] + bd_ref[...])
    p1 = jax.nn.leaky_relu(hfc @ Wp1_ref[...] + bp1_ref[...])
    v1 = jax.nn.leaky_relu(hfc @ Wv1_ref[...] + bv1_ref[...])
    logits = p1 @ Wp2_ref[...] + bp2_ref[...]
    logits_ref[...] = logits
    policy_ref[...] = jax.nn.softmax(logits, axis=1)
    value_ref[...] = jnp.tanh(v1 @ Wv2_ref[...] + bv2_ref[...])


def kernel(node_type, num_inverted_predecessors, edge_index, batch, seq_embedding,
           emb_table, W1, b1, W2, b2, Wd, bd, Wp1, bp1, Wp2, bp2, Wv1, bv1, Wv2, bv2):
    src = edge_index[0]
    dst = edge_index[1]
    pad = NP - N
    ntp = jnp.pad(node_type, (0, pad))
    ninvp = jnp.pad(num_inverted_predecessors, (0, pad))
    batchp = jnp.pad(batch, (0, pad), constant_values=G)

    zeros_n = jnp.zeros((NP,), jnp.float32)
    zeros_n8 = jnp.zeros((NP, 8), jnp.float32)
    ones_w = jnp.ones((W,), jnp.float32)

    k9 = jnp.arange(9)
    table9 = jnp.concatenate(
        [emb_table[k9 // 3], (k9 % 3).astype(jnp.float32)[:, None]], axis=1)
    A16 = jnp.zeros((16, 32), jnp.float32).at[:9].set(table9 @ W1)

    deg0, deg1 = _deg_call()(dst, ones_w, zeros_n)

    R = NP // 128
    dinv2d, cns2d = pl.pallas_call(
        _prep_body,
        out_shape=(jax.ShapeDtypeStruct((R, 128), jnp.float32),
                   jax.ShapeDtypeStruct((R, 128), jnp.int32)),
    )(deg0.reshape(R, 128), deg1.reshape(R, 128),
      ntp.reshape(R, 128), ninvp.reshape(R, 128))
    dinv = dinv2d.reshape(NP)
    cns = cns2d.reshape(NP)

    t0, t1 = _tacc_call()(src, dst, dinv, cns, zeros_n)

    BLK = 2176
    nblk = NP // BLK
    dinv_c = dinv.reshape(NP, 1)
    cns_c = cns.reshape(NP, 1)
    h1, qa, qb, qc, qd = pl.pallas_call(
        _h1q_body,
        grid=(nblk,),
        in_specs=[
            pl.BlockSpec((BLK, 9), lambda i: (i, 0)),
            pl.BlockSpec((BLK, 9), lambda i: (i, 0)),
            pl.BlockSpec((BLK, 1), lambda i: (i, 0)),
            pl.BlockSpec((BLK, 1), lambda i: (i, 0)),
            pl.BlockSpec((16, 32), lambda i: (0, 0)),
            pl.BlockSpec((1, 32), lambda i: (0, 0)),
        ],
        out_specs=[
            pl.BlockSpec((BLK, 32), lambda i: (i, 0)),
            pl.BlockSpec((BLK, 8), lambda i: (i, 0)),
            pl.BlockSpec((BLK, 8), lambda i: (i, 0)),
            pl.BlockSpec((BLK, 8), lambda i: (i, 0)),
            pl.BlockSpec((BLK, 8), lambda i: (i, 0)),
        ],
        out_shape=(jax.ShapeDtypeStruct((NP, 32), jnp.float32),
                   jax.ShapeDtypeStruct((NP, 8), jnp.float32),
                   jax.ShapeDtypeStruct((NP, 8), jnp.float32),
                   jax.ShapeDtypeStruct((NP, 8), jnp.float32),
                   jax.ShapeDtypeStruct((NP, 8), jnp.float32)),
    )(t0.reshape(NP, 9), t1.reshape(NP, 9), dinv_c, cns_c,
      A16, b1.reshape(1, 32))

    za, zb = _zacc_call()(src, dst, qa, qb, zeros_n8)
    zc, zd = _zacc_call()(src, dst, qc, qd, zeros_n8)

    aig = pl.pallas_call(
        _pool_body,
        grid=(nblk,),
        in_specs=[
            pl.BlockSpec((BLK, 8), lambda i: (i, 0)),
            pl.BlockSpec((BLK, 8), lambda i: (i, 0)),
            pl.BlockSpec((BLK, 8), lambda i: (i, 0)),
            pl.BlockSpec((BLK, 8), lambda i: (i, 0)),
            pl.BlockSpec((BLK, 32), lambda i: (i, 0)),
            pl.BlockSpec((BLK, 1), lambda i: (i, 0)),
            pl.BlockSpec((BLK, 1), lambda i: (i, 0)),
            pl.BlockSpec((32, 32), lambda i: (0, 0)),
            pl.BlockSpec((1, 32), lambda i: (0, 0)),
        ],
        out_specs=pl.BlockSpec((G, 64), lambda i: (0, 0)),
        out_shape=jax.ShapeDtypeStruct((G, 64), jnp.float32),
        scratch_shapes=[
            pltpu.VMEM((G, 32), jnp.float32),
            pltpu.VMEM((G, 32), jnp.float32),
            pltpu.VMEM((G, 32), jnp.float32),
        ],
    )(za, zb, zc, zd, h1, dinv_c, batchp.reshape(NP, 1), W2, b2.reshape(1, 32))

    final = jnp.concatenate([aig, seq_embedding], axis=1)

    logits, policy, value = pl.pallas_call(
        _head_body,
        out_shape=(
            jax.ShapeDtypeStruct((G, 7), jnp.float32),
            jax.ShapeDtypeStruct((G, 7), jnp.float32),
            jax.ShapeDtypeStruct((G, 1), jnp.float32),
        ),
    )(final, Wd, bd, Wp1, bp1, Wp2, bp2, Wv1, bv1, Wv2, bv2)
    return (logits, policy, value.reshape(-1), final, aig)


# speedup vs baseline: 71.8535x; 1.1250x over previous
import functools

import jax
import jax.numpy as jnp
from jax import lax
from jax.experimental import pallas as pl
from jax.experimental.pallas import tpu as pltpu
from jax.experimental.pallas import tpu_sc as plsc

N = 100000
NP = 100096
E = 6400000
G = 16
CHN = NP // 16
W = 2000
WZ = 800

_MESH = plsc.VectorSubcoreMesh(core_axis_name="c", subcore_axis_name="s")


def _deg_body(dst_hbm, ones_hbm, zeros_hbm, out0_hbm, out1_hbm,
              idx_v, ones_v, bz_v, deg_sh, sem):
    cid = lax.axis_index("c")
    sid = lax.axis_index("s")
    pltpu.sync_copy(zeros_hbm.at[pl.ds(sid * CHN, CHN)], bz_v)
    pltpu.sync_copy(bz_v, deg_sh.at[pl.ds(sid * CHN, CHN)])
    pltpu.sync_copy(ones_hbm, ones_v)
    plsc.subcore_barrier()

    chunk = E // 32
    base0 = cid * (E // 2) + sid * chunk

    def wbody(w, _):
        base = base0 + w * W
        pltpu.sync_copy(dst_hbm.at[pl.ds(base, W)], idx_v)
        pltpu.sync_copy(ones_v, deg_sh.at[idx_v], add=True)
        return 0

    lax.fori_loop(0, chunk // W, wbody, 0)
    plsc.subcore_barrier()

    pltpu.sync_copy(deg_sh.at[pl.ds(sid * CHN, CHN)], bz_v)

    @pl.when(cid == 0)
    def _():
        pltpu.sync_copy(bz_v, out0_hbm.at[pl.ds(sid * CHN, CHN)])

    @pl.when(cid == 1)
    def _():
        pltpu.sync_copy(bz_v, out1_hbm.at[pl.ds(sid * CHN, CHN)])


_deg_call = functools.partial(
    pl.kernel, _deg_body,
    out_type=(jax.ShapeDtypeStruct((NP,), jnp.float32),
              jax.ShapeDtypeStruct((NP,), jnp.float32)),
    mesh=_MESH,
    compiler_params=pltpu.CompilerParams(use_tc_tiling_on_sc=False),
    scratch_types=[
        pltpu.VMEM((W,), jnp.int32),
        pltpu.VMEM((W,), jnp.float32),
        pltpu.VMEM((CHN,), jnp.float32),
        pltpu.VMEM_SHARED((NP,), jnp.float32),
        pltpu.SemaphoreType.DMA,
    ],
)


def _tacc_body(src_hbm, dst_hbm, dinv_hbm, cns_hbm, zeros_hbm, out0_hbm, out1_hbm,
               sidx_v, didx_v, dval_v, cval_v, idx2_v, bzf_v, bdf_v, bdi_v,
               dinv_sh, cns_sh, t_sh, sem1, sem2):
    cid = lax.axis_index("c")
    sid = lax.axis_index("s")
    pltpu.sync_copy(dinv_hbm.at[pl.ds(sid * CHN, CHN)], bdf_v)
    pltpu.sync_copy(bdf_v, dinv_sh.at[pl.ds(sid * CHN, CHN)])
    pltpu.sync_copy(cns_hbm.at[pl.ds(sid * CHN, CHN)], bdi_v)
    pltpu.sync_copy(bdi_v, cns_sh.at[pl.ds(sid * CHN, CHN)])
    pltpu.sync_copy(zeros_hbm.at[pl.ds(0, CHN)], bzf_v)
    for k in range(9):
        pltpu.sync_copy(bzf_v, t_sh.at[pl.ds(sid * (CHN * 9) + k * CHN, CHN)])
    plsc.subcore_barrier()

    chunk = E // 32
    base0 = cid * (E // 2) + sid * chunk

    def wbody(w, _):
        base = base0 + w * W
        pltpu.sync_copy(src_hbm.at[pl.ds(base, W)], sidx_v)
        pltpu.sync_copy(dst_hbm.at[pl.ds(base, W)], didx_v)
        g1 = pltpu.async_copy(dinv_sh.at[sidx_v], dval_v, sem1)
        g2 = pltpu.async_copy(cns_sh.at[sidx_v], cval_v, sem2)
        g1.wait()
        g2.wait()

        def jbody(j, _):
            sl = pl.ds(j * 16, 16)
            idx2_v[sl] = didx_v[sl] * 9 + cval_v[sl]
            return 0

        lax.fori_loop(0, W // 16, jbody, 0)
        pltpu.sync_copy(dval_v, t_sh.at[idx2_v], add=True)
        return 0

    lax.fori_loop(0, chunk // W, wbody, 0)
    plsc.subcore_barrier()

    for k in range(9):
        off = sid * (CHN * 9) + k * CHN
        pltpu.sync_copy(t_sh.at[pl.ds(off, CHN)], bzf_v)

        @pl.when(cid == 0)
        def _():
            pltpu.sync_copy(bzf_v, out0_hbm.at[pl.ds(off, CHN)])

        @pl.when(cid == 1)
        def _():
            pltpu.sync_copy(bzf_v, out1_hbm.at[pl.ds(off, CHN)])


_tacc_call = functools.partial(
    pl.kernel, _tacc_body,
    out_type=(jax.ShapeDtypeStruct((NP * 9,), jnp.float32),
              jax.ShapeDtypeStruct((NP * 9,), jnp.float32)),
    mesh=_MESH,
    compiler_params=pltpu.CompilerParams(use_tc_tiling_on_sc=False),
    scratch_types=[
        pltpu.VMEM((W,), jnp.int32),
        pltpu.VMEM((W,), jnp.int32),
        pltpu.VMEM((W,), jnp.float32),
        pltpu.VMEM((W,), jnp.int32),
        pltpu.VMEM((W,), jnp.int32),
        pltpu.VMEM((CHN,), jnp.float32),
        pltpu.VMEM((CHN,), jnp.float32),
        pltpu.VMEM((CHN,), jnp.int32),
        pltpu.VMEM_SHARED((NP,), jnp.float32),
        pltpu.VMEM_SHARED((NP,), jnp.int32),
        pltpu.VMEM_SHARED((NP * 9,), jnp.float32),
        pltpu.SemaphoreType.DMA,
        pltpu.SemaphoreType.DMA,
    ],
)


def _zacc_body(src_hbm, dst_hbm, qL_hbm, qR_hbm, zeros_hbm, outL_hbm, outR_hbm,
               sidx_v, didx_v, rows_v, bq_v, z_sh, sem):
    cid = lax.axis_index("c")
    sid = lax.axis_index("s")
    ZB = CHN // 8
    pltpu.sync_copy(zeros_hbm, bq_v)
    for k in range(8):
        off = sid * CHN + k * ZB
        pltpu.sync_copy(bq_v, z_sh.at[pl.ds(off, ZB)])
    plsc.subcore_barrier()

    chunk = E // 16
    base0 = sid * chunk

    def wbody(w, _):
        base = base0 + w * WZ
        pltpu.sync_copy(src_hbm.at[pl.ds(base, WZ)], sidx_v)
        pltpu.sync_copy(dst_hbm.at[pl.ds(base, WZ)], didx_v)

        @pl.when(cid == 0)
        def _():
            pltpu.sync_copy(qL_hbm.at[sidx_v], rows_v)

        @pl.when(cid == 1)
        def _():
            pltpu.sync_copy(qR_hbm.at[sidx_v], rows_v)

        pltpu.sync_copy(rows_v, z_sh.at[didx_v], add=True)
        return 0

    lax.fori_loop(0, chunk // WZ, wbody, 0)
    plsc.subcore_barrier()

    for k in range(8):
        off = sid * CHN + k * ZB
        pltpu.sync_copy(z_sh.at[pl.ds(off, ZB)], bq_v)

        @pl.when(cid == 0)
        def _():
            pltpu.sync_copy(bq_v, outL_hbm.at[pl.ds(off, ZB)])

        @pl.when(cid == 1)
        def _():
            pltpu.sync_copy(bq_v, outR_hbm.at[pl.ds(off, ZB)])


_zacc_call = functools.partial(
    pl.kernel, _zacc_body,
    out_type=(jax.ShapeDtypeStruct((NP, 16), jnp.float32),
              jax.ShapeDtypeStruct((NP, 16), jnp.float32)),
    mesh=_MESH,
    compiler_params=pltpu.CompilerParams(use_tc_tiling_on_sc=False),
    scratch_types=[
        pltpu.VMEM((WZ,), jnp.int32),
        pltpu.VMEM((WZ,), jnp.int32),
        pltpu.VMEM((WZ, 16), jnp.float32),
        pltpu.VMEM((CHN // 8, 16), jnp.float32),
        pltpu.VMEM_SHARED((NP, 16), jnp.float32),
        pltpu.SemaphoreType.DMA,
    ],
)


def _prep_body(deg0_ref, deg1_ref, nt_ref, ninv_ref, dinv_ref, cns_ref):
    deg = deg0_ref[...] + deg1_ref[...] + 1.0
    dinv_ref[...] = lax.rsqrt(deg)
    cns_ref[...] = nt_ref[...] * 3 + ninv_ref[...]


def _h1q_body(t0_ref, t1_ref, dinv_ref, cns_ref, A_ref, b1_ref,
              h1_ref, ql_ref, qr_ref):
    dinv = dinv_ref[...]
    t = t0_ref[...] + t1_ref[...]
    onehot = (lax.broadcasted_iota(jnp.int32, t.shape, 1) == cns_ref[...]
              ).astype(jnp.float32)
    T9 = dinv * t + (dinv * dinv) * onehot
    T = jnp.concatenate([T9, jnp.zeros((T9.shape[0], 7), jnp.float32)], axis=1)
    h1 = jax.nn.relu(jnp.dot(T, A_ref[...], preferred_element_type=jnp.float32)
                     + b1_ref[...])
    h1_ref[...] = h1
    q = h1 * dinv
    ql_ref[...] = q[:, 0:16]
    qr_ref[...] = q[:, 16:32]


def _pool_body(zl_ref, zr_ref, h1_ref, dinv_ref, batch_ref,
               W2_ref, b2_ref, aig_ref, sum_acc, cnt_acc, max_acc):
    i = pl.program_id(0)

    @pl.when(i == 0)
    def _():
        sum_acc[...] = jnp.zeros_like(sum_acc)
        cnt_acc[...] = jnp.zeros_like(cnt_acc)
        max_acc[...] = jnp.full_like(max_acc, -jnp.inf)

    dinv = dinv_ref[...]
    h1 = h1_ref[...]
    z = jnp.concatenate([zl_ref[...], zr_ref[...]], axis=1)
    u = dinv * z + (dinv * dinv) * h1
    h2 = jnp.dot(u, W2_ref[...], preferred_element_type=jnp.float32) + b2_ref[...]
    b = batch_ref[...]
    onehot = (lax.broadcasted_iota(jnp.int32, (h2.shape[0], G), 1) == b
              ).astype(jnp.float32)
    sum_acc[...] += jax.lax.dot_general(
        onehot, h2, (((0,), (0,)), ((), ())), preferred_element_type=jnp.float32)
    cnt_acc[...] += jnp.sum(onehot, axis=0)[:, None]
    for g in range(G):
        m = jnp.where(b == g, h2, -jnp.inf)
        max_acc[g, :] = jnp.maximum(max_acc[g, :], jnp.max(m, axis=0))

    @pl.when(i == pl.num_programs(0) - 1)
    def _():
        mean = sum_acc[...] / jnp.maximum(cnt_acc[...], 1.0)
        aig = jnp.concatenate([mean, max_acc[...]], axis=1)
        aig_ref[...] = jnp.round(aig * 1000.0) / 1000.0


def _head_body(final_ref, Wd_ref, bd_ref, Wp1_ref, bp1_ref, Wp2_ref, bp2_ref,
               Wv1_ref, bv1_ref, Wv2_ref, bv2_ref,
               logits_ref, policy_ref, value_ref):
    final = final_ref[...]
    hfc = jax.nn.leaky_relu(final @ Wd_ref[...] + bd_ref[...])
    p1 = jax.nn.leaky_relu(hfc @ Wp1_ref[...] + bp1_ref[...])
    v1 = jax.nn.leaky_relu(hfc @ Wv1_ref[...] + bv1_ref[...])
    logits = p1 @ Wp2_ref[...] + bp2_ref[...]
    logits_ref[...] = logits
    policy_ref[...] = jax.nn.softmax(logits, axis=1)
    value_ref[...] = jnp.tanh(v1 @ Wv2_ref[...] + bv2_ref[...])


def kernel(node_type, num_inverted_predecessors, edge_index, batch, seq_embedding,
           emb_table, W1, b1, W2, b2, Wd, bd, Wp1, bp1, Wp2, bp2, Wv1, bv1, Wv2, bv2):
    src = edge_index[0]
    dst = edge_index[1]
    pad = NP - N
    ntp = jnp.pad(node_type, (0, pad))
    ninvp = jnp.pad(num_inverted_predecessors, (0, pad))
    batchp = jnp.pad(batch, (0, pad), constant_values=G)

    zeros_n = jnp.zeros((NP,), jnp.float32)
    zeros_zb = jnp.zeros((CHN // 8, 16), jnp.float32)
    ones_w = jnp.ones((W,), jnp.float32)

    k9 = jnp.arange(9)
    table9 = jnp.concatenate(
        [emb_table[k9 // 3], (k9 % 3).astype(jnp.float32)[:, None]], axis=1)
    A16 = jnp.zeros((16, 32), jnp.float32).at[:9].set(table9 @ W1)

    deg0, deg1 = _deg_call()(dst, ones_w, zeros_n)

    R = NP // 128
    dinv2d, cns2d = pl.pallas_call(
        _prep_body,
        out_shape=(jax.ShapeDtypeStruct((R, 128), jnp.float32),
                   jax.ShapeDtypeStruct((R, 128), jnp.int32)),
    )(deg0.reshape(R, 128), deg1.reshape(R, 128),
      ntp.reshape(R, 128), ninvp.reshape(R, 128))
    dinv = dinv2d.reshape(NP)
    cns = cns2d.reshape(NP)

    t0, t1 = _tacc_call()(src, dst, dinv, cns, zeros_n)

    BLK = 2176
    nblk = NP // BLK
    dinv_c = dinv.reshape(NP, 1)
    cns_c = cns.reshape(NP, 1)
    h1, ql, qr = pl.pallas_call(
        _h1q_body,
        grid=(nblk,),
        in_specs=[
            pl.BlockSpec((BLK, 9), lambda i: (i, 0)),
            pl.BlockSpec((BLK, 9), lambda i: (i, 0)),
            pl.BlockSpec((BLK, 1), lambda i: (i, 0)),
            pl.BlockSpec((BLK, 1), lambda i: (i, 0)),
            pl.BlockSpec((16, 32), lambda i: (0, 0)),
            pl.BlockSpec((1, 32), lambda i: (0, 0)),
        ],
        out_specs=[
            pl.BlockSpec((BLK, 32), lambda i: (i, 0)),
            pl.BlockSpec((BLK, 16), lambda i: (i, 0)),
            pl.BlockSpec((BLK, 16), lambda i: (i, 0)),
        ],
        out_shape=(jax.ShapeDtypeStruct((NP, 32), jnp.float32),
                   jax.ShapeDtypeStruct((NP, 16), jnp.float32),
                   jax.ShapeDtypeStruct((NP, 16), jnp.float32)),
    )(t0.reshape(NP, 9), t1.reshape(NP, 9), dinv_c, cns_c,
      A16, b1.reshape(1, 32))

    zl, zr = _zacc_call()(src, dst, ql, qr, zeros_zb)

    aig = pl.pallas_call(
        _pool_body,
        grid=(nblk,),
        in_specs=[
            pl.BlockSpec((BLK, 16), lambda i: (i, 0)),
            pl.BlockSpec((BLK, 16), lambda i: (i, 0)),
            pl.BlockSpec((BLK, 32), lambda i: (i, 0)),
            pl.BlockSpec((BLK, 1), lambda i: (i, 0)),
            pl.BlockSpec((BLK, 1), lambda i: (i, 0)),
            pl.BlockSpec((32, 32), lambda i: (0, 0)),
            pl.BlockSpec((1, 32), lambda i: (0, 0)),
        ],
        out_specs=pl.BlockSpec((G, 64), lambda i: (0, 0)),
        out_shape=jax.ShapeDtypeStruct((G, 64), jnp.float32),
        scratch_shapes=[
            pltpu.VMEM((G, 32), jnp.float32),
            pltpu.VMEM((G, 32), jnp.float32),
            pltpu.VMEM((G, 32), jnp.float32),
        ],
    )(zl, zr, h1, dinv_c, batchp.reshape(NP, 1), W2, b2.reshape(1, 32))

    final = jnp.concatenate([aig, seq_embedding], axis=1)

    logits, policy, value = pl.pallas_call(
        _head_body,
        out_shape=(
            jax.ShapeDtypeStruct((G, 7), jnp.float32),
            jax.ShapeDtypeStruct((G, 7), jnp.float32),
            jax.ShapeDtypeStruct((G, 1), jnp.float32),
        ),
    )(final, Wd, bd, Wp1, bp1, Wp2, bp2, Wv1, bv1, Wv2, bv2)
    return (logits, policy, value.reshape(-1), final, aig)


# speedup vs baseline: 76.0336x; 1.0582x over previous
import functools

import jax
import jax.numpy as jnp
from jax import lax
from jax.experimental import pallas as pl
from jax.experimental.pallas import tpu as pltpu
from jax.experimental.pallas import tpu_sc as plsc

N = 100000
NP = 100096
E = 6400000
G = 16
CHN = NP // 16
W = 2000
WZ = 1000

_MESH = plsc.VectorSubcoreMesh(core_axis_name="c", subcore_axis_name="s")


def _deg_body(dst_hbm, ones_hbm, zeros_hbm, out0_hbm, out1_hbm,
              idx_v, ones_v, bz_v, deg_sh, sem):
    cid = lax.axis_index("c")
    sid = lax.axis_index("s")
    pltpu.sync_copy(zeros_hbm.at[pl.ds(sid * CHN, CHN)], bz_v)
    pltpu.sync_copy(bz_v, deg_sh.at[pl.ds(sid * CHN, CHN)])
    pltpu.sync_copy(ones_hbm, ones_v)
    plsc.subcore_barrier()

    chunk = E // 32
    base0 = cid * (E // 2) + sid * chunk

    def wbody(w, _):
        base = base0 + w * W
        pltpu.sync_copy(dst_hbm.at[pl.ds(base, W)], idx_v)
        pltpu.sync_copy(ones_v, deg_sh.at[idx_v], add=True)
        return 0

    lax.fori_loop(0, chunk // W, wbody, 0)
    plsc.subcore_barrier()

    pltpu.sync_copy(deg_sh.at[pl.ds(sid * CHN, CHN)], bz_v)

    @pl.when(cid == 0)
    def _():
        pltpu.sync_copy(bz_v, out0_hbm.at[pl.ds(sid * CHN, CHN)])

    @pl.when(cid == 1)
    def _():
        pltpu.sync_copy(bz_v, out1_hbm.at[pl.ds(sid * CHN, CHN)])


_deg_call = functools.partial(
    pl.kernel, _deg_body,
    out_type=(jax.ShapeDtypeStruct((NP,), jnp.float32),
              jax.ShapeDtypeStruct((NP,), jnp.float32)),
    mesh=_MESH,
    compiler_params=pltpu.CompilerParams(use_tc_tiling_on_sc=False),
    scratch_types=[
        pltpu.VMEM((W,), jnp.int32),
        pltpu.VMEM((W,), jnp.float32),
        pltpu.VMEM((CHN,), jnp.float32),
        pltpu.VMEM_SHARED((NP,), jnp.float32),
        pltpu.SemaphoreType.DMA,
    ],
)


def _tacc_body(src_hbm, dst_hbm, dinv_hbm, cns_hbm, zeros_hbm, out0_hbm, out1_hbm,
               sidx_v, didx_v, dval_v, cval_v, idx2_v, bzf_v, bdf_v, bdi_v,
               dinv_sh, cns_sh, t_sh, sem1, sem2):
    cid = lax.axis_index("c")
    sid = lax.axis_index("s")
    pltpu.sync_copy(dinv_hbm.at[pl.ds(sid * CHN, CHN)], bdf_v)
    pltpu.sync_copy(bdf_v, dinv_sh.at[pl.ds(sid * CHN, CHN)])
    pltpu.sync_copy(cns_hbm.at[pl.ds(sid * CHN, CHN)], bdi_v)
    pltpu.sync_copy(bdi_v, cns_sh.at[pl.ds(sid * CHN, CHN)])
    pltpu.sync_copy(zeros_hbm.at[pl.ds(0, CHN)], bzf_v)
    for k in range(9):
        pltpu.sync_copy(bzf_v, t_sh.at[pl.ds(sid * (CHN * 9) + k * CHN, CHN)])
    plsc.subcore_barrier()

    chunk = E // 32
    base0 = cid * (E // 2) + sid * chunk

    def wbody(w, _):
        base = base0 + w * W
        pltpu.sync_copy(src_hbm.at[pl.ds(base, W)], sidx_v)
        pltpu.sync_copy(dst_hbm.at[pl.ds(base, W)], didx_v)
        g1 = pltpu.async_copy(dinv_sh.at[sidx_v], dval_v, sem1)
        g2 = pltpu.async_copy(cns_sh.at[sidx_v], cval_v, sem2)
        g1.wait()
        g2.wait()

        def jbody(j, _):
            sl = pl.ds(j * 16, 16)
            idx2_v[sl] = didx_v[sl] * 9 + cval_v[sl]
            return 0

        lax.fori_loop(0, W // 16, jbody, 0)
        pltpu.sync_copy(dval_v, t_sh.at[idx2_v], add=True)
        return 0

    lax.fori_loop(0, chunk // W, wbody, 0)
    plsc.subcore_barrier()

    for k in range(9):
        off = sid * (CHN * 9) + k * CHN
        pltpu.sync_copy(t_sh.at[pl.ds(off, CHN)], bzf_v)

        @pl.when(cid == 0)
        def _():
            pltpu.sync_copy(bzf_v, out0_hbm.at[pl.ds(off, CHN)])

        @pl.when(cid == 1)
        def _():
            pltpu.sync_copy(bzf_v, out1_hbm.at[pl.ds(off, CHN)])


_tacc_call = functools.partial(
    pl.kernel, _tacc_body,
    out_type=(jax.ShapeDtypeStruct((NP * 9,), jnp.float32),
              jax.ShapeDtypeStruct((NP * 9,), jnp.float32)),
    mesh=_MESH,
    compiler_params=pltpu.CompilerParams(use_tc_tiling_on_sc=False),
    scratch_types=[
        pltpu.VMEM((W,), jnp.int32),
        pltpu.VMEM((W,), jnp.int32),
        pltpu.VMEM((W,), jnp.float32),
        pltpu.VMEM((W,), jnp.int32),
        pltpu.VMEM((W,), jnp.int32),
        pltpu.VMEM((CHN,), jnp.float32),
        pltpu.VMEM((CHN,), jnp.float32),
        pltpu.VMEM((CHN,), jnp.int32),
        pltpu.VMEM_SHARED((NP,), jnp.float32),
        pltpu.VMEM_SHARED((NP,), jnp.int32),
        pltpu.VMEM_SHARED((NP * 9,), jnp.float32),
        pltpu.SemaphoreType.DMA,
        pltpu.SemaphoreType.DMA,
    ],
)


def _zacc_body(src_hbm, dst_hbm, qL_hbm, qR_hbm, zeros_hbm, outL_hbm, outR_hbm,
               sidx_v, didx_v, rows_v, bq_v, z_sh, sem):
    cid = lax.axis_index("c")
    sid = lax.axis_index("s")
    ZB = CHN // 8
    pltpu.sync_copy(zeros_hbm, bq_v)
    for k in range(8):
        off = sid * CHN + k * ZB
        pltpu.sync_copy(bq_v, z_sh.at[pl.ds(off, ZB)])
    plsc.subcore_barrier()

    chunk = E // 16
    base0 = sid * chunk

    def wbody(w, _):
        base = base0 + w * WZ
        pltpu.sync_copy(src_hbm.at[pl.ds(base, WZ)], sidx_v)
        pltpu.sync_copy(dst_hbm.at[pl.ds(base, WZ)], didx_v)

        @pl.when(cid == 0)
        def _():
            pltpu.sync_copy(qL_hbm.at[sidx_v], rows_v)

        @pl.when(cid == 1)
        def _():
            pltpu.sync_copy(qR_hbm.at[sidx_v], rows_v)

        pltpu.sync_copy(rows_v, z_sh.at[didx_v], add=True)
        return 0

    lax.fori_loop(0, chunk // WZ, wbody, 0)
    plsc.subcore_barrier()

    for k in range(8):
        off = sid * CHN + k * ZB
        pltpu.sync_copy(z_sh.at[pl.ds(off, ZB)], bq_v)

        @pl.when(cid == 0)
        def _():
            pltpu.sync_copy(bq_v, outL_hbm.at[pl.ds(off, ZB)])

        @pl.when(cid == 1)
        def _():
            pltpu.sync_copy(bq_v, outR_hbm.at[pl.ds(off, ZB)])


_zacc_call = functools.partial(
    pl.kernel, _zacc_body,
    out_type=(jax.ShapeDtypeStruct((NP, 16), jnp.float32),
              jax.ShapeDtypeStruct((NP, 16), jnp.float32)),
    mesh=_MESH,
    compiler_params=pltpu.CompilerParams(use_tc_tiling_on_sc=False),
    scratch_types=[
        pltpu.VMEM((WZ,), jnp.int32),
        pltpu.VMEM((WZ,), jnp.int32),
        pltpu.VMEM((WZ, 16), jnp.float32),
        pltpu.VMEM((CHN // 8, 16), jnp.float32),
        pltpu.VMEM_SHARED((NP, 16), jnp.float32),
        pltpu.SemaphoreType.DMA,
    ],
)


def _prep_body(deg0_ref, deg1_ref, nt_ref, ninv_ref, dinv_ref, cns_ref):
    deg = deg0_ref[...] + deg1_ref[...] + 1.0
    dinv_ref[...] = lax.rsqrt(deg)
    cns_ref[...] = nt_ref[...] * 3 + ninv_ref[...]


def _h1q_body(t0_ref, t1_ref, dinv_ref, cns_ref, A_ref, b1_ref,
              h1_ref, ql_ref, qr_ref):
    dinv = dinv_ref[...]
    t = t0_ref[...] + t1_ref[...]
    onehot = (lax.broadcasted_iota(jnp.int32, t.shape, 1) == cns_ref[...]
              ).astype(jnp.float32)
    T9 = dinv * t + (dinv * dinv) * onehot
    T = jnp.concatenate([T9, jnp.zeros((T9.shape[0], 7), jnp.float32)], axis=1)
    h1 = jax.nn.relu(jnp.dot(T, A_ref[...], preferred_element_type=jnp.float32)
                     + b1_ref[...])
    h1_ref[...] = h1
    q = h1 * dinv
    ql_ref[...] = q[:, 0:16]
    qr_ref[...] = q[:, 16:32]


def _pool_body(zl_ref, zr_ref, h1_ref, dinv_ref, batch_ref,
               W2_ref, b2_ref, aig_ref, sum_acc, cnt_acc, max_acc):
    i = pl.program_id(0)

    @pl.when(i == 0)
    def _():
        sum_acc[...] = jnp.zeros_like(sum_acc)
        cnt_acc[...] = jnp.zeros_like(cnt_acc)
        max_acc[...] = jnp.full_like(max_acc, -jnp.inf)

    dinv = dinv_ref[...]
    h1 = h1_ref[...]
    z = jnp.concatenate([zl_ref[...], zr_ref[...]], axis=1)
    u = dinv * z + (dinv * dinv) * h1
    h2 = jnp.dot(u, W2_ref[...], preferred_element_type=jnp.float32) + b2_ref[...]
    b = batch_ref[...]
    onehot = (lax.broadcasted_iota(jnp.int32, (h2.shape[0], G), 1) == b
              ).astype(jnp.float32)
    sum_acc[...] += jax.lax.dot_general(
        onehot, h2, (((0,), (0,)), ((), ())), preferred_element_type=jnp.float32)
    cnt_acc[...] += jnp.sum(onehot, axis=0)[:, None]
    for g in range(G):
        m = jnp.where(b == g, h2, -jnp.inf)
        max_acc[g, :] = jnp.maximum(max_acc[g, :], jnp.max(m, axis=0))

    @pl.when(i == pl.num_programs(0) - 1)
    def _():
        mean = sum_acc[...] / jnp.maximum(cnt_acc[...], 1.0)
        aig = jnp.concatenate([mean, max_acc[...]], axis=1)
        aig_ref[...] = jnp.round(aig * 1000.0) / 1000.0


def _head_body(final_ref, Wd_ref, bd_ref, Wp1_ref, bp1_ref, Wp2_ref, bp2_ref,
               Wv1_ref, bv1_ref, Wv2_ref, bv2_ref,
               logits_ref, policy_ref, value_ref):
    final = final_ref[...]
    hfc = jax.nn.leaky_relu(final @ Wd_ref[...] + bd_ref[...])
    p1 = jax.nn.leaky_relu(hfc @ Wp1_ref[...] + bp1_ref[...])
    v1 = jax.nn.leaky_relu(hfc @ Wv1_ref[...] + bv1_ref[...])
    logits = p1 @ Wp2_ref[...] + bp2_ref[...]
    logits_ref[...] = logits
    policy_ref[...] = jax.nn.softmax(logits, axis=1)
    value_ref[...] = jnp.tanh(v1 @ Wv2_ref[...] + bv2_ref[...])


def kernel(node_type, num_inverted_predecessors, edge_index, batch, seq_embedding,
           emb_table, W1, b1, W2, b2, Wd, bd, Wp1, bp1, Wp2, bp2, Wv1, bv1, Wv2, bv2):
    src = edge_index[0]
    dst = edge_index[1]
    pad = NP - N
    ntp = jnp.pad(node_type, (0, pad))
    ninvp = jnp.pad(num_inverted_predecessors, (0, pad))
    batchp = jnp.pad(batch, (0, pad), constant_values=G)

    zeros_n = jnp.zeros((NP,), jnp.float32)
    zeros_zb = jnp.zeros((CHN // 8, 16), jnp.float32)
    ones_w = jnp.ones((W,), jnp.float32)

    k9 = jnp.arange(9)
    table9 = jnp.concatenate(
        [emb_table[k9 // 3], (k9 % 3).astype(jnp.float32)[:, None]], axis=1)
    A16 = jnp.zeros((16, 32), jnp.float32).at[:9].set(table9 @ W1)

    deg0, deg1 = _deg_call()(dst, ones_w, zeros_n)

    R = NP // 128
    dinv2d, cns2d = pl.pallas_call(
        _prep_body,
        out_shape=(jax.ShapeDtypeStruct((R, 128), jnp.float32),
                   jax.ShapeDtypeStruct((R, 128), jnp.int32)),
    )(deg0.reshape(R, 128), deg1.reshape(R, 128),
      ntp.reshape(R, 128), ninvp.reshape(R, 128))
    dinv = dinv2d.reshape(NP)
    cns = cns2d.reshape(NP)

    t0, t1 = _tacc_call()(src, dst, dinv, cns, zeros_n)

    BLK = 2176
    nblk = NP // BLK
    dinv_c = dinv.reshape(NP, 1)
    cns_c = cns.reshape(NP, 1)
    h1, ql, qr = pl.pallas_call(
        _h1q_body,
        grid=(nblk,),
        in_specs=[
            pl.BlockSpec((BLK, 9), lambda i: (i, 0)),
            pl.BlockSpec((BLK, 9), lambda i: (i, 0)),
            pl.BlockSpec((BLK, 1), lambda i: (i, 0)),
            pl.BlockSpec((BLK, 1), lambda i: (i, 0)),
            pl.BlockSpec((16, 32), lambda i: (0, 0)),
            pl.BlockSpec((1, 32), lambda i: (0, 0)),
        ],
        out_specs=[
            pl.BlockSpec((BLK, 32), lambda i: (i, 0)),
            pl.BlockSpec((BLK, 16), lambda i: (i, 0)),
            pl.BlockSpec((BLK, 16), lambda i: (i, 0)),
        ],
        out_shape=(jax.ShapeDtypeStruct((NP, 32), jnp.float32),
                   jax.ShapeDtypeStruct((NP, 16), jnp.float32),
                   jax.ShapeDtypeStruct((NP, 16), jnp.float32)),
    )(t0.reshape(NP, 9), t1.reshape(NP, 9), dinv_c, cns_c,
      A16, b1.reshape(1, 32))

    zl, zr = _zacc_call()(src, dst, ql, qr, zeros_zb)

    aig = pl.pallas_call(
        _pool_body,
        grid=(nblk,),
        in_specs=[
            pl.BlockSpec((BLK, 16), lambda i: (i, 0)),
            pl.BlockSpec((BLK, 16), lambda i: (i, 0)),
            pl.BlockSpec((BLK, 32), lambda i: (i, 0)),
            pl.BlockSpec((BLK, 1), lambda i: (i, 0)),
            pl.BlockSpec((BLK, 1), lambda i: (i, 0)),
            pl.BlockSpec((32, 32), lambda i: (0, 0)),
            pl.BlockSpec((1, 32), lambda i: (0, 0)),
        ],
        out_specs=pl.BlockSpec((G, 64), lambda i: (0, 0)),
        out_shape=jax.ShapeDtypeStruct((G, 64), jnp.float32),
        scratch_shapes=[
            pltpu.VMEM((G, 32), jnp.float32),
            pltpu.VMEM((G, 32), jnp.float32),
            pltpu.VMEM((G, 32), jnp.float32),
        ],
    )(zl, zr, h1, dinv_c, batchp.reshape(NP, 1), W2, b2.reshape(1, 32))

    final = jnp.concatenate([aig, seq_embedding], axis=1)

    logits, policy, value = pl.pallas_call(
        _head_body,
        out_shape=(
            jax.ShapeDtypeStruct((G, 7), jnp.float32),
            jax.ShapeDtypeStruct((G, 7), jnp.float32),
            jax.ShapeDtypeStruct((G, 1), jnp.float32),
        ),
    )(final, Wd, bd, Wp1, bp1, Wp2, bp2, Wv1, bv1, Wv2, bv2)
    return (logits, policy, value.reshape(-1), final, aig)


# speedup vs baseline: 77.8954x; 1.0245x over previous
import functools

import jax
import jax.numpy as jnp
from jax import lax
from jax.experimental import pallas as pl
from jax.experimental.pallas import tpu as pltpu
from jax.experimental.pallas import tpu_sc as plsc

N = 100000
NP = 100096
E = 6400000
G = 16
CHN = NP // 16
W = 2000
WZ = 400

_MESH = plsc.VectorSubcoreMesh(core_axis_name="c", subcore_axis_name="s")


def _deg_body(dst_hbm, ones_hbm, zeros_hbm, out0_hbm, out1_hbm,
              idx_v, ones_v, bz_v, deg_sh, sem):
    cid = lax.axis_index("c")
    sid = lax.axis_index("s")
    pltpu.sync_copy(zeros_hbm.at[pl.ds(sid * CHN, CHN)], bz_v)
    pltpu.sync_copy(bz_v, deg_sh.at[pl.ds(sid * CHN, CHN)])
    pltpu.sync_copy(ones_hbm, ones_v)
    plsc.subcore_barrier()

    chunk = E // 32
    base0 = cid * (E // 2) + sid * chunk

    def wbody(w, _):
        base = base0 + w * W
        pltpu.sync_copy(dst_hbm.at[pl.ds(base, W)], idx_v)
        pltpu.sync_copy(ones_v, deg_sh.at[idx_v], add=True)
        return 0

    lax.fori_loop(0, chunk // W, wbody, 0)
    plsc.subcore_barrier()

    pltpu.sync_copy(deg_sh.at[pl.ds(sid * CHN, CHN)], bz_v)

    @pl.when(cid == 0)
    def _():
        pltpu.sync_copy(bz_v, out0_hbm.at[pl.ds(sid * CHN, CHN)])

    @pl.when(cid == 1)
    def _():
        pltpu.sync_copy(bz_v, out1_hbm.at[pl.ds(sid * CHN, CHN)])


_deg_call = functools.partial(
    pl.kernel, _deg_body,
    out_type=(jax.ShapeDtypeStruct((NP,), jnp.float32),
              jax.ShapeDtypeStruct((NP,), jnp.float32)),
    mesh=_MESH,
    compiler_params=pltpu.CompilerParams(use_tc_tiling_on_sc=False),
    scratch_types=[
        pltpu.VMEM((W,), jnp.int32),
        pltpu.VMEM((W,), jnp.float32),
        pltpu.VMEM((CHN,), jnp.float32),
        pltpu.VMEM_SHARED((NP,), jnp.float32),
        pltpu.SemaphoreType.DMA,
    ],
)


def _tacc_body(src_hbm, dst_hbm, dinv_hbm, cns_hbm, zeros_hbm, out0_hbm, out1_hbm,
               sidx_v, didx_v, dval_v, cval_v, idx2_v, bzf_v, bdf_v, bdi_v,
               dinv_sh, cns_sh, t_sh, sem1, sem2):
    cid = lax.axis_index("c")
    sid = lax.axis_index("s")
    pltpu.sync_copy(dinv_hbm.at[pl.ds(sid * CHN, CHN)], bdf_v)
    pltpu.sync_copy(bdf_v, dinv_sh.at[pl.ds(sid * CHN, CHN)])
    pltpu.sync_copy(cns_hbm.at[pl.ds(sid * CHN, CHN)], bdi_v)
    pltpu.sync_copy(bdi_v, cns_sh.at[pl.ds(sid * CHN, CHN)])
    pltpu.sync_copy(zeros_hbm.at[pl.ds(0, CHN)], bzf_v)
    for k in range(9):
        pltpu.sync_copy(bzf_v, t_sh.at[pl.ds(sid * (CHN * 9) + k * CHN, CHN)])
    plsc.subcore_barrier()

    chunk = E // 32
    base0 = cid * (E // 2) + sid * chunk

    def wbody(w, _):
        base = base0 + w * W
        pltpu.sync_copy(src_hbm.at[pl.ds(base, W)], sidx_v)
        pltpu.sync_copy(dst_hbm.at[pl.ds(base, W)], didx_v)
        g1 = pltpu.async_copy(dinv_sh.at[sidx_v], dval_v, sem1)
        g2 = pltpu.async_copy(cns_sh.at[sidx_v], cval_v, sem2)
        g1.wait()
        g2.wait()

        def jbody(j, _):
            sl = pl.ds(j * 16, 16)
            idx2_v[sl] = didx_v[sl] * 9 + cval_v[sl]
            return 0

        lax.fori_loop(0, W // 16, jbody, 0)
        pltpu.sync_copy(dval_v, t_sh.at[idx2_v], add=True)
        return 0

    lax.fori_loop(0, chunk // W, wbody, 0)
    plsc.subcore_barrier()

    for k in range(9):
        off = sid * (CHN * 9) + k * CHN
        pltpu.sync_copy(t_sh.at[pl.ds(off, CHN)], bzf_v)

        @pl.when(cid == 0)
        def _():
            pltpu.sync_copy(bzf_v, out0_hbm.at[pl.ds(off, CHN)])

        @pl.when(cid == 1)
        def _():
            pltpu.sync_copy(bzf_v, out1_hbm.at[pl.ds(off, CHN)])


_tacc_call = functools.partial(
    pl.kernel, _tacc_body,
    out_type=(jax.ShapeDtypeStruct((NP * 9,), jnp.float32),
              jax.ShapeDtypeStruct((NP * 9,), jnp.float32)),
    mesh=_MESH,
    compiler_params=pltpu.CompilerParams(use_tc_tiling_on_sc=False),
    scratch_types=[
        pltpu.VMEM((W,), jnp.int32),
        pltpu.VMEM((W,), jnp.int32),
        pltpu.VMEM((W,), jnp.float32),
        pltpu.VMEM((W,), jnp.int32),
        pltpu.VMEM((W,), jnp.int32),
        pltpu.VMEM((CHN,), jnp.float32),
        pltpu.VMEM((CHN,), jnp.float32),
        pltpu.VMEM((CHN,), jnp.int32),
        pltpu.VMEM_SHARED((NP,), jnp.float32),
        pltpu.VMEM_SHARED((NP,), jnp.int32),
        pltpu.VMEM_SHARED((NP * 9,), jnp.float32),
        pltpu.SemaphoreType.DMA,
        pltpu.SemaphoreType.DMA,
    ],
)


def _zacc_body(src_hbm, dst_hbm, qL_hbm, qR_hbm, zeros_hbm, outL_hbm, outR_hbm,
               sidx0_v, didx0_v, sidx1_v, didx1_v, rows0_v, rows1_v, bq_v,
               z_sh, sem0, sem1):
    cid = lax.axis_index("c")
    sid = lax.axis_index("s")
    ZB = CHN // 16
    pltpu.sync_copy(zeros_hbm, bq_v)
    for k in range(16):
        off = sid * CHN + k * ZB
        pltpu.sync_copy(bq_v, z_sh.at[pl.ds(off, ZB)])
    plsc.subcore_barrier()

    chunk = E // 16
    base0 = sid * chunk
    npairs = chunk // (2 * WZ)

    def run(q_hbm):
        pltpu.sync_copy(src_hbm.at[pl.ds(base0, WZ)], sidx0_v)
        pltpu.sync_copy(dst_hbm.at[pl.ds(base0, WZ)], didx0_v)
        pltpu.async_copy(q_hbm.at[sidx0_v], rows0_v, sem0)

        def pbody(p, _):
            base_b = base0 + (2 * p + 1) * WZ
            pltpu.sync_copy(src_hbm.at[pl.ds(base_b, WZ)], sidx1_v)
            pltpu.sync_copy(dst_hbm.at[pl.ds(base_b, WZ)], didx1_v)
            pltpu.async_copy(q_hbm.at[sidx1_v], rows1_v, sem1)

            pltpu.make_async_copy(q_hbm.at[sidx0_v], rows0_v, sem0).wait()
            pltpu.sync_copy(rows0_v, z_sh.at[didx0_v], add=True)

            @pl.when(p < npairs - 1)
            def _():
                base_a = base0 + (2 * p + 2) * WZ
                pltpu.sync_copy(src_hbm.at[pl.ds(base_a, WZ)], sidx0_v)
                pltpu.sync_copy(dst_hbm.at[pl.ds(base_a, WZ)], didx0_v)
                pltpu.async_copy(q_hbm.at[sidx0_v], rows0_v, sem0)

            pltpu.make_async_copy(q_hbm.at[sidx1_v], rows1_v, sem1).wait()
            pltpu.sync_copy(rows1_v, z_sh.at[didx1_v], add=True)
            return 0

        lax.fori_loop(0, npairs, pbody, 0)

    @pl.when(cid == 0)
    def _():
        run(qL_hbm)

    @pl.when(cid == 1)
    def _():
        run(qR_hbm)

    plsc.subcore_barrier()

    for k in range(16):
        off = sid * CHN + k * ZB
        pltpu.sync_copy(z_sh.at[pl.ds(off, ZB)], bq_v)

        @pl.when(cid == 0)
        def _():
            pltpu.sync_copy(bq_v, outL_hbm.at[pl.ds(off, ZB)])

        @pl.when(cid == 1)
        def _():
            pltpu.sync_copy(bq_v, outR_hbm.at[pl.ds(off, ZB)])


_zacc_call = functools.partial(
    pl.kernel, _zacc_body,
    out_type=(jax.ShapeDtypeStruct((NP, 16), jnp.float32),
              jax.ShapeDtypeStruct((NP, 16), jnp.float32)),
    mesh=_MESH,
    compiler_params=pltpu.CompilerParams(use_tc_tiling_on_sc=False),
    scratch_types=[
        pltpu.VMEM((WZ,), jnp.int32),
        pltpu.VMEM((WZ,), jnp.int32),
        pltpu.VMEM((WZ,), jnp.int32),
        pltpu.VMEM((WZ,), jnp.int32),
        pltpu.VMEM((WZ, 16), jnp.float32),
        pltpu.VMEM((WZ, 16), jnp.float32),
        pltpu.VMEM((CHN // 16, 16), jnp.float32),
        pltpu.VMEM_SHARED((NP, 16), jnp.float32),
        pltpu.SemaphoreType.DMA,
        pltpu.SemaphoreType.DMA,
    ],
)


def _prep_body(deg0_ref, deg1_ref, nt_ref, ninv_ref, dinv_ref, cns_ref):
    deg = deg0_ref[...] + deg1_ref[...] + 1.0
    dinv_ref[...] = lax.rsqrt(deg)
    cns_ref[...] = nt_ref[...] * 3 + ninv_ref[...]


def _h1q_body(t0_ref, t1_ref, dinv_ref, cns_ref, A_ref, b1_ref,
              h1_ref, ql_ref, qr_ref):
    dinv = dinv_ref[...]
    t = t0_ref[...] + t1_ref[...]
    onehot = (lax.broadcasted_iota(jnp.int32, t.shape, 1) == cns_ref[...]
              ).astype(jnp.float32)
    T9 = dinv * t + (dinv * dinv) * onehot
    T = jnp.concatenate([T9, jnp.zeros((T9.shape[0], 7), jnp.float32)], axis=1)
    h1 = jax.nn.relu(jnp.dot(T, A_ref[...], preferred_element_type=jnp.float32)
                     + b1_ref[...])
    h1_ref[...] = h1
    q = h1 * dinv
    ql_ref[...] = q[:, 0:16]
    qr_ref[...] = q[:, 16:32]


def _pool_body(zl_ref, zr_ref, h1_ref, dinv_ref, batch_ref,
               W2_ref, b2_ref, aig_ref, sum_acc, cnt_acc, max_acc):
    i = pl.program_id(0)

    @pl.when(i == 0)
    def _():
        sum_acc[...] = jnp.zeros_like(sum_acc)
        cnt_acc[...] = jnp.zeros_like(cnt_acc)
        max_acc[...] = jnp.full_like(max_acc, -jnp.inf)

    dinv = dinv_ref[...]
    h1 = h1_ref[...]
    z = jnp.concatenate([zl_ref[...], zr_ref[...]], axis=1)
    u = dinv * z + (dinv * dinv) * h1
    h2 = jnp.dot(u, W2_ref[...], preferred_element_type=jnp.float32) + b2_ref[...]
    b = batch_ref[...]
    onehot = (lax.broadcasted_iota(jnp.int32, (h2.shape[0], G), 1) == b
              ).astype(jnp.float32)
    sum_acc[...] += jax.lax.dot_general(
        onehot, h2, (((0,), (0,)), ((), ())), preferred_element_type=jnp.float32)
    cnt_acc[...] += jnp.sum(onehot, axis=0)[:, None]
    for g in range(G):
        m = jnp.where(b == g, h2, -jnp.inf)
        max_acc[g, :] = jnp.maximum(max_acc[g, :], jnp.max(m, axis=0))

    @pl.when(i == pl.num_programs(0) - 1)
    def _():
        mean = sum_acc[...] / jnp.maximum(cnt_acc[...], 1.0)
        aig = jnp.concatenate([mean, max_acc[...]], axis=1)
        aig_ref[...] = jnp.round(aig * 1000.0) / 1000.0


def _head_body(final_ref, Wd_ref, bd_ref, Wp1_ref, bp1_ref, Wp2_ref, bp2_ref,
               Wv1_ref, bv1_ref, Wv2_ref, bv2_ref,
               logits_ref, policy_ref, value_ref):
    final = final_ref[...]
    hfc = jax.nn.leaky_relu(final @ Wd_ref[...] + bd_ref[...])
    p1 = jax.nn.leaky_relu(hfc @ Wp1_ref[...] + bp1_ref[...])
    v1 = jax.nn.leaky_relu(hfc @ Wv1_ref[...] + bv1_ref[...])
    logits = p1 @ Wp2_ref[...] + bp2_ref[...]
    logits_ref[...] = logits
    policy_ref[...] = jax.nn.softmax(logits, axis=1)
    value_ref[...] = jnp.tanh(v1 @ Wv2_ref[...] + bv2_ref[...])


def kernel(node_type, num_inverted_predecessors, edge_index, batch, seq_embedding,
           emb_table, W1, b1, W2, b2, Wd, bd, Wp1, bp1, Wp2, bp2, Wv1, bv1, Wv2, bv2):
    src = edge_index[0]
    dst = edge_index[1]
    pad = NP - N
    ntp = jnp.pad(node_type, (0, pad))
    ninvp = jnp.pad(num_inverted_predecessors, (0, pad))
    batchp = jnp.pad(batch, (0, pad), constant_values=G)

    zeros_n = jnp.zeros((NP,), jnp.float32)
    zeros_zb = jnp.zeros((CHN // 16, 16), jnp.float32)
    ones_w = jnp.ones((W,), jnp.float32)

    k9 = jnp.arange(9)
    table9 = jnp.concatenate(
        [emb_table[k9 // 3], (k9 % 3).astype(jnp.float32)[:, None]], axis=1)
    A16 = jnp.zeros((16, 32), jnp.float32).at[:9].set(table9 @ W1)

    deg0, deg1 = _deg_call()(dst, ones_w, zeros_n)

    R = NP // 128
    dinv2d, cns2d = pl.pallas_call(
        _prep_body,
        out_shape=(jax.ShapeDtypeStruct((R, 128), jnp.float32),
                   jax.ShapeDtypeStruct((R, 128), jnp.int32)),
    )(deg0.reshape(R, 128), deg1.reshape(R, 128),
      ntp.reshape(R, 128), ninvp.reshape(R, 128))
    dinv = dinv2d.reshape(NP)
    cns = cns2d.reshape(NP)

    t0, t1 = _tacc_call()(src, dst, dinv, cns, zeros_n)

    BLK = 2176
    nblk = NP // BLK
    dinv_c = dinv.reshape(NP, 1)
    cns_c = cns.reshape(NP, 1)
    h1, ql, qr = pl.pallas_call(
        _h1q_body,
        grid=(nblk,),
        in_specs=[
            pl.BlockSpec((BLK, 9), lambda i: (i, 0)),
            pl.BlockSpec((BLK, 9), lambda i: (i, 0)),
            pl.BlockSpec((BLK, 1), lambda i: (i, 0)),
            pl.BlockSpec((BLK, 1), lambda i: (i, 0)),
            pl.BlockSpec((16, 32), lambda i: (0, 0)),
            pl.BlockSpec((1, 32), lambda i: (0, 0)),
        ],
        out_specs=[
            pl.BlockSpec((BLK, 32), lambda i: (i, 0)),
            pl.BlockSpec((BLK, 16), lambda i: (i, 0)),
            pl.BlockSpec((BLK, 16), lambda i: (i, 0)),
        ],
        out_shape=(jax.ShapeDtypeStruct((NP, 32), jnp.float32),
                   jax.ShapeDtypeStruct((NP, 16), jnp.float32),
                   jax.ShapeDtypeStruct((NP, 16), jnp.float32)),
    )(t0.reshape(NP, 9), t1.reshape(NP, 9), dinv_c, cns_c,
      A16, b1.reshape(1, 32))

    zl, zr = _zacc_call()(src, dst, ql, qr, zeros_zb)

    aig = pl.pallas_call(
        _pool_body,
        grid=(nblk,),
        in_specs=[
            pl.BlockSpec((BLK, 16), lambda i: (i, 0)),
            pl.BlockSpec((BLK, 16), lambda i: (i, 0)),
            pl.BlockSpec((BLK, 32), lambda i: (i, 0)),
            pl.BlockSpec((BLK, 1), lambda i: (i, 0)),
            pl.BlockSpec((BLK, 1), lambda i: (i, 0)),
            pl.BlockSpec((32, 32), lambda i: (0, 0)),
            pl.BlockSpec((1, 32), lambda i: (0, 0)),
        ],
        out_specs=pl.BlockSpec((G, 64), lambda i: (0, 0)),
        out_shape=jax.ShapeDtypeStruct((G, 64), jnp.float32),
        scratch_shapes=[
            pltpu.VMEM((G, 32), jnp.float32),
            pltpu.VMEM((G, 32), jnp.float32),
            pltpu.VMEM((G, 32), jnp.float32),
        ],
    )(zl, zr, h1, dinv_c, batchp.reshape(NP, 1), W2, b2.reshape(1, 32))

    final = jnp.concatenate([aig, seq_embedding], axis=1)

    logits, policy, value = pl.pallas_call(
        _head_body,
        out_shape=(
            jax.ShapeDtypeStruct((G, 7), jnp.float32),
            jax.ShapeDtypeStruct((G, 7), jnp.float32),
            jax.ShapeDtypeStruct((G, 1), jnp.float32),
        ),
    )(final, Wd, bd, Wp1, bp1, Wp2, bp2, Wv1, bv1, Wv2, bv2)
    return (logits, policy, value.reshape(-1), final, aig)
